# baseline (speedup 1.0000x reference)
import functools

import jax
import jax.numpy as jnp
from jax import lax
from jax.experimental import pallas as pl
from jax.experimental.pallas import tpu as pltpu
from jax.experimental.pallas import tpu_sc as plsc


def _last_relevant_sc(lstm_flat, seqlens, B, T, D):
    mesh = plsc.VectorSubcoreMesh(core_axis_name="c", subcore_axis_name="s")

    @functools.partial(
        pl.kernel,
        mesh=mesh,
        out_type=jax.ShapeDtypeStruct((B, D), jnp.float32),
        scratch_types=[
            pltpu.VMEM((B,), jnp.int32),
            pltpu.VMEM((B,), jnp.int32),
            pltpu.VMEM((B, D), jnp.float32),
            pltpu.SemaphoreType.DMA,
        ],
    )
    def body(lstm_hbm, seq_hbm, out_hbm, seq_v, idx_v, rows_v, sem):
        cid = lax.axis_index("c")
        sid = lax.axis_index("s")

        @pl.when(jnp.logical_and(cid == 0, sid == 0))
        def _():
            pltpu.sync_copy(seq_hbm, seq_v)
            idx_v[...] = lax.iota(jnp.int32, B) * T + seq_v[...] - 1
            pltpu.async_copy(lstm_hbm.at[idx_v], rows_v, sem).wait()
            pltpu.sync_copy(rows_v, out_hbm)

    return body(lstm_flat, seqlens)


def kernel(lstm, seqlens):
    B, T, D = lstm.shape
    lstm_flat = lstm.reshape(B * T, D)
    return _last_relevant_sc(lstm_flat, seqlens, B, T, D)


# speedup vs baseline: 1.0663x; 1.0663x over previous
import functools

import jax
import jax.numpy as jnp
from jax import lax
from jax.experimental import pallas as pl
from jax.experimental.pallas import tpu as pltpu
from jax.experimental.pallas import tpu_sc as plsc


def _last_relevant_sc(lstm_flat, seqlens, B, T, D):
    mesh = plsc.VectorSubcoreMesh(
        core_axis_name="c", subcore_axis_name="s", num_cores=1
    )

    @functools.partial(
        pl.kernel,
        mesh=mesh,
        out_type=jax.ShapeDtypeStruct((B, D), jnp.float32),
        scratch_types=[
            pltpu.VMEM((B,), jnp.int32),
            pltpu.VMEM((B, D), jnp.float32),
            pltpu.SemaphoreType.DMA,
        ],
    )
    def body(lstm_hbm, seq_hbm, out_hbm, seq_v, rows_v, sem):
        cid = lax.axis_index("c")
        sid = lax.axis_index("s")

        @pl.when(jnp.logical_and(cid == 0, sid == 0))
        def _():
            pltpu.sync_copy(seq_hbm, seq_v)
            idx = lax.iota(jnp.int32, B) * T + seq_v[...] - 1
            pltpu.async_copy(lstm_hbm.at[idx], rows_v, sem).wait()
            pltpu.sync_copy(rows_v, out_hbm)

    return body(lstm_flat, seqlens)


def kernel(lstm, seqlens):
    B, T, D = lstm.shape
    lstm_flat = lstm.reshape(B * T, D)
    return _last_relevant_sc(lstm_flat, seqlens, B, T, D)


# speedup vs baseline: 1.1082x; 1.0393x over previous
import functools

import jax
import jax.numpy as jnp
from jax.experimental import pallas as pl
from jax.experimental.pallas import tpu as pltpu
from jax.experimental.pallas import tpu_sc as plsc


def _last_relevant_scs(lstm, seqlens, B, T, D):
    mesh = plsc.ScalarSubcoreMesh(axis_name="c", num_cores=1)

    @functools.partial(
        pl.kernel,
        mesh=mesh,
        out_type=jax.ShapeDtypeStruct((B, D), jnp.float32),
        scratch_types=[
            pltpu.SMEM((B,), jnp.int32),
            pltpu.SemaphoreType.DMA,
        ],
    )
    def body(lstm_hbm, seq_hbm, out_hbm, seq_s, sem):
        pltpu.sync_copy(seq_hbm, seq_s)
        copies = []
        for b in range(B):
            t = seq_s[b] - 1
            copies.append(pltpu.async_copy(lstm_hbm.at[b, t], out_hbm.at[b], sem))
        for c in copies:
            c.wait()

    return body(lstm, seqlens)


def kernel(lstm, seqlens):
    B, T, D = lstm.shape
    return _last_relevant_scs(lstm, seqlens, B, T, D)
